# Initial kernel scaffold; baseline (speedup 1.0000x reference)
import functools

import jax
import jax.numpy as jnp
from jax import lax
from jax.experimental import pallas as pl
from jax.experimental.pallas import tpu as pltpu
from jax.experimental.pallas import tpu_sc as plsc

N = 10000
E = 320000
D_IN = 128
D_HID = 16
N_CLS = 41

NC = 2
NS = 16
NW = NC * NS
EC = 128
K = 80
E_PAD = NW * K * EC
NP = 10112
RPT = NP // NS
D2 = 48

_mesh = plsc.VectorSubcoreMesh(core_axis_name="c", subcore_axis_name="s")


def _make_deg_kernel():
    @functools.partial(
        pl.kernel,
        mesh=_mesh,
        out_type=jax.ShapeDtypeStruct((NC, NP, 16), jnp.float32),
        scratch_types=[
            pltpu.VMEM((K, EC), jnp.int32),
            pltpu.VMEM((EC, 16), jnp.float32),
            pltpu.VMEM_SHARED((NP, 16), jnp.float32),
        ],
    )
    def deg_kernel(dst_hbm, ones_hbm, zeros_hbm, out_hbm, dst_v, ones_v, acc):
        c = lax.axis_index("c")
        s = lax.axis_index("s")
        tid = c * NS + s
        r0 = s * RPT
        pltpu.sync_copy(zeros_hbm.at[pl.ds(r0, RPT), :], acc.at[pl.ds(r0, RPT), :])
        pltpu.sync_copy(ones_hbm, ones_v)
        pltpu.sync_copy(dst_hbm.at[tid], dst_v)
        plsc.subcore_barrier()

        def body(j, carry):
            pltpu.sync_copy(ones_v, acc.at[dst_v.at[j]], add=True)
            return carry

        lax.fori_loop(0, K, body, 0)
        plsc.subcore_barrier()
        pltpu.sync_copy(acc.at[pl.ds(r0, RPT), :], out_hbm.at[c, pl.ds(r0, RPT), :])

    return deg_kernel


def _make_edge_kernel(d):
    @functools.partial(
        pl.kernel,
        mesh=_mesh,
        out_type=jax.ShapeDtypeStruct((NC, NP, d), jnp.float32),
        scratch_types=[
            pltpu.VMEM((K, EC), jnp.int32),
            pltpu.VMEM((K, EC), jnp.int32),
            pltpu.VMEM((EC, d), jnp.float32),
            pltpu.VMEM_SHARED((NP, d), jnp.float32),
            pltpu.SemaphoreType.DMA,
        ],
    )
    def edge_kernel(hs_hbm, src_hbm, dst_hbm, zeros_hbm, out_hbm,
                    src_v, dst_v, rows_v, acc, sem):
        c = lax.axis_index("c")
        s = lax.axis_index("s")
        tid = c * NS + s
        r0 = s * RPT
        @pl.when(c == 0)
        def _():
            pltpu.sync_copy(hs_hbm.at[pl.ds(r0, RPT), :], acc.at[pl.ds(r0, RPT), :])

        @pl.when(c != 0)
        def _():
            pltpu.sync_copy(zeros_hbm.at[pl.ds(r0, RPT), :], acc.at[pl.ds(r0, RPT), :])

        pltpu.sync_copy(src_hbm.at[tid], src_v)
        pltpu.sync_copy(dst_hbm.at[tid], dst_v)
        plsc.subcore_barrier()

        def body(j, carry):
            pltpu.async_copy(hs_hbm.at[src_v.at[j]], rows_v, sem).wait()
            pltpu.sync_copy(rows_v, acc.at[dst_v.at[j]], add=True)
            return carry

        lax.fori_loop(0, K, body, 0)
        plsc.subcore_barrier()
        pltpu.sync_copy(acc.at[pl.ds(r0, RPT), :], out_hbm.at[c, pl.ds(r0, RPT), :])

    return edge_kernel


_deg_kernel = _make_deg_kernel()
_edge_kernel16 = _make_edge_kernel(D_HID)
_edge_kernel48 = _make_edge_kernel(D2)


def _tc_stage1(feats_ref, w1_ref, degp_ref, hs1_ref, dinv_ref):
    deg = degp_ref[0] + degp_ref[1] + 1.0
    dinv = lax.rsqrt(deg)
    dinv_ref[...] = dinv
    h = jnp.dot(feats_ref[...], w1_ref[...], preferred_element_type=jnp.float32)
    hs1_ref[...] = h * dinv[:, 0:1]


def _tc_stage2(p_ref, dinv_ref, b1_ref, w2_ref, hs2_ref):
    dinv1 = dinv_ref[:, 0:1]
    h1 = jnp.maximum((p_ref[0] + p_ref[1]) * dinv1 + b1_ref[...], 0.0)
    hs2_ref[...] = jnp.dot(h1, w2_ref[...], preferred_element_type=jnp.float32) * dinv1


def _tc_stage3(p_ref, dinv_ref, b2_ref, out_ref):
    out_ref[...] = (p_ref[0] + p_ref[1]) * dinv_ref[:, 0:1] + b2_ref[...]


def kernel(feats, edge_index, W1, b1, W2, b2):
    f32 = jnp.float32
    feats_p = jnp.pad(feats, ((0, NP - N), (0, 0)))
    src = jnp.pad(edge_index[0], (0, E_PAD - E))
    dst = jnp.pad(edge_index[1], (0, E_PAD - E), constant_values=N)
    src_t = src.reshape(NW, K, EC)
    dst_t = dst.reshape(NW, K, EC)
    ones16 = jnp.ones((EC, 16), f32)
    zeros16 = jnp.zeros((NP, 16), f32)
    zeros48 = jnp.zeros((NP, D2), f32)
    w2p = jnp.pad(W2, ((0, 0), (0, D2 - N_CLS)))
    b1r = b1.reshape(1, D_HID)
    b2r = jnp.pad(b2, (0, D2 - N_CLS)).reshape(1, D2)

    degp = _deg_kernel(dst_t, ones16, zeros16)

    hs1, dinv = pl.pallas_call(
        _tc_stage1,
        out_shape=[
            jax.ShapeDtypeStruct((NP, D_HID), f32),
            jax.ShapeDtypeStruct((NP, 16), f32),
        ],
    )(feats_p, W1, degp)

    p1 = _edge_kernel16(hs1, src_t, dst_t, zeros16)

    hs2 = pl.pallas_call(
        _tc_stage2,
        out_shape=jax.ShapeDtypeStruct((NP, D2), f32),
    )(p1, dinv, b1r, w2p)

    p2 = _edge_kernel48(hs2, src_t, dst_t, zeros48)

    out = pl.pallas_call(
        _tc_stage3,
        out_shape=jax.ShapeDtypeStruct((NP, D2), f32),
    )(p2, dinv, b2r)

    return out[:N, :N_CLS]


# speedup vs baseline: 21.3574x; 21.3574x over previous
import functools

import jax
import jax.numpy as jnp
from jax import lax
from jax.experimental import pallas as pl
from jax.experimental.pallas import tpu as pltpu
from jax.experimental.pallas import tpu_sc as plsc

N = 10000
E = 320000
D_IN = 128
D_HID = 16
N_CLS = 41

NC = 2
NS = 16
NW = NC * NS
EC = 128
K = 80
E_PAD = NW * K * EC
NP = 10112
RPT = NP // NS
D2 = 48

_mesh = plsc.VectorSubcoreMesh(core_axis_name="c", subcore_axis_name="s")


def _make_deg_kernel():
    @functools.partial(
        pl.kernel,
        mesh=_mesh,
        compiler_params=pltpu.CompilerParams(use_tc_tiling_on_sc=False),
        out_type=jax.ShapeDtypeStruct((NC, NP, 16), jnp.float32),
        scratch_types=[
            pltpu.VMEM((K, EC), jnp.int32),
            pltpu.VMEM((EC, 16), jnp.float32),
            pltpu.VMEM_SHARED((NP, 16), jnp.float32),
        ],
    )
    def deg_kernel(dst_hbm, ones_hbm, zeros_hbm, out_hbm, dst_v, ones_v, acc):
        c = lax.axis_index("c")
        s = lax.axis_index("s")
        tid = c * NS + s
        r0 = s * RPT
        pltpu.sync_copy(zeros_hbm.at[pl.ds(r0, RPT), :], acc.at[pl.ds(r0, RPT), :])
        pltpu.sync_copy(ones_hbm, ones_v)
        pltpu.sync_copy(dst_hbm.at[tid], dst_v)
        plsc.subcore_barrier()

        def body(j, carry):
            pltpu.sync_copy(ones_v, acc.at[dst_v.at[j]], add=True)
            return carry

        lax.fori_loop(0, K, body, 0)
        plsc.subcore_barrier()
        pltpu.sync_copy(acc.at[pl.ds(r0, RPT), :], out_hbm.at[c, pl.ds(r0, RPT), :])

    return deg_kernel


def _make_edge_kernel(d):
    @functools.partial(
        pl.kernel,
        mesh=_mesh,
        compiler_params=pltpu.CompilerParams(use_tc_tiling_on_sc=False),
        out_type=jax.ShapeDtypeStruct((NC, NP, d), jnp.float32),
        scratch_types=[
            pltpu.VMEM((K, EC), jnp.int32),
            pltpu.VMEM((K, EC), jnp.int32),
            pltpu.VMEM((EC, d), jnp.float32),
            pltpu.VMEM_SHARED((NP, d), jnp.float32),
            pltpu.SemaphoreType.DMA,
        ],
    )
    def edge_kernel(hs_hbm, src_hbm, dst_hbm, zeros_hbm, out_hbm,
                    src_v, dst_v, rows_v, acc, sem):
        c = lax.axis_index("c")
        s = lax.axis_index("s")
        tid = c * NS + s
        r0 = s * RPT
        @pl.when(c == 0)
        def _():
            pltpu.sync_copy(hs_hbm.at[pl.ds(r0, RPT), :], acc.at[pl.ds(r0, RPT), :])

        @pl.when(c != 0)
        def _():
            pltpu.sync_copy(zeros_hbm.at[pl.ds(r0, RPT), :], acc.at[pl.ds(r0, RPT), :])

        pltpu.sync_copy(src_hbm.at[tid], src_v)
        pltpu.sync_copy(dst_hbm.at[tid], dst_v)
        plsc.subcore_barrier()

        def body(j, carry):
            pltpu.async_copy(hs_hbm.at[src_v.at[j]], rows_v, sem).wait()
            pltpu.sync_copy(rows_v, acc.at[dst_v.at[j]], add=True)
            return carry

        lax.fori_loop(0, K, body, 0)
        plsc.subcore_barrier()
        pltpu.sync_copy(acc.at[pl.ds(r0, RPT), :], out_hbm.at[c, pl.ds(r0, RPT), :])

    return edge_kernel


_deg_kernel = _make_deg_kernel()
_edge_kernel16 = _make_edge_kernel(D_HID)
_edge_kernel48 = _make_edge_kernel(D2)


def _tc_stage1(feats_ref, w1_ref, degp_ref, hs1_ref, dinv_ref):
    deg = degp_ref[0] + degp_ref[1] + 1.0
    dinv = lax.rsqrt(deg)
    dinv_ref[...] = dinv
    h = jnp.dot(feats_ref[...], w1_ref[...], preferred_element_type=jnp.float32)
    hs1_ref[...] = h * dinv[:, 0:1]


def _tc_stage2(p_ref, dinv_ref, b1_ref, w2_ref, hs2_ref):
    dinv1 = dinv_ref[:, 0:1]
    h1 = jnp.maximum((p_ref[0] + p_ref[1]) * dinv1 + b1_ref[...], 0.0)
    hs2_ref[...] = jnp.dot(h1, w2_ref[...], preferred_element_type=jnp.float32) * dinv1


def _tc_stage3(p_ref, dinv_ref, b2_ref, out_ref):
    out_ref[...] = (p_ref[0] + p_ref[1]) * dinv_ref[:, 0:1] + b2_ref[...]


def kernel(feats, edge_index, W1, b1, W2, b2):
    f32 = jnp.float32
    feats_p = jnp.pad(feats, ((0, NP - N), (0, 0)))
    src = jnp.pad(edge_index[0], (0, E_PAD - E))
    dst = jnp.pad(edge_index[1], (0, E_PAD - E), constant_values=N)
    src_t = src.reshape(NW, K, EC)
    dst_t = dst.reshape(NW, K, EC)
    ones16 = jnp.ones((EC, 16), f32)
    zeros16 = jnp.zeros((NP, 16), f32)
    zeros48 = jnp.zeros((NP, D2), f32)
    w2p = jnp.pad(W2, ((0, 0), (0, D2 - N_CLS)))
    b1r = b1.reshape(1, D_HID)
    b2r = jnp.pad(b2, (0, D2 - N_CLS)).reshape(1, D2)

    degp = _deg_kernel(dst_t, ones16, zeros16)

    hs1, dinv = pl.pallas_call(
        _tc_stage1,
        out_shape=[
            jax.ShapeDtypeStruct((NP, D_HID), f32),
            jax.ShapeDtypeStruct((NP, 16), f32),
        ],
    )(feats_p, W1, degp)

    p1 = _edge_kernel16(hs1, src_t, dst_t, zeros16)

    hs2 = pl.pallas_call(
        _tc_stage2,
        out_shape=jax.ShapeDtypeStruct((NP, D2), f32),
    )(p1, dinv, b1r, w2p)

    p2 = _edge_kernel48(hs2, src_t, dst_t, zeros48)

    out = pl.pallas_call(
        _tc_stage3,
        out_shape=jax.ShapeDtypeStruct((NP, D2), f32),
    )(p2, dinv, b2r)

    return out[:N, :N_CLS]


# speedup vs baseline: 39.1417x; 1.8327x over previous
import functools

import jax
import jax.numpy as jnp
from jax import lax
from jax.experimental import pallas as pl
from jax.experimental.pallas import tpu as pltpu
from jax.experimental.pallas import tpu_sc as plsc

N = 10000
E = 320000
D_IN = 128
D_HID = 16
N_CLS = 41

NC = 2
NS = 16
NW = NC * NS
EC = 128
K = 80
E_PAD = NW * K * EC
NP = 10112
RPT = NP // NS
D2 = 48

_mesh = plsc.VectorSubcoreMesh(core_axis_name="c", subcore_axis_name="s")


def _make_deg_kernel():
    @functools.partial(
        pl.kernel,
        mesh=_mesh,
        compiler_params=pltpu.CompilerParams(use_tc_tiling_on_sc=False),
        out_type=jax.ShapeDtypeStruct((NC, NP, 16), jnp.float32),
        scratch_types=[
            pltpu.VMEM((K, EC), jnp.int32),
            pltpu.VMEM((EC, 16), jnp.float32),
            pltpu.VMEM_SHARED((NP, 16), jnp.float32),
        ],
    )
    def deg_kernel(dst_hbm, ones_hbm, zeros_hbm, out_hbm, dst_v, ones_v, acc):
        c = lax.axis_index("c")
        s = lax.axis_index("s")
        tid = c * NS + s
        r0 = s * RPT
        pltpu.sync_copy(zeros_hbm.at[pl.ds(r0, RPT), :], acc.at[pl.ds(r0, RPT), :])
        pltpu.sync_copy(ones_hbm, ones_v)
        pltpu.sync_copy(dst_hbm.at[tid], dst_v)
        plsc.subcore_barrier()

        def body(j, carry):
            pltpu.sync_copy(ones_v, acc.at[dst_v.at[j]], add=True)
            return carry

        lax.fori_loop(0, K, body, 0)
        plsc.subcore_barrier()
        pltpu.sync_copy(acc.at[pl.ds(r0, RPT), :], out_hbm.at[c, pl.ds(r0, RPT), :])

    return deg_kernel


def _make_edge_kernel(d):
    GG = 4
    G = K // GG

    @functools.partial(
        pl.kernel,
        mesh=_mesh,
        compiler_params=pltpu.CompilerParams(use_tc_tiling_on_sc=False),
        out_type=jax.ShapeDtypeStruct((NC, NP, d), jnp.float32),
        scratch_types=[
            pltpu.VMEM((K, EC), jnp.int32),
            pltpu.VMEM((K, EC), jnp.int32),
            [pltpu.VMEM((EC, d), jnp.float32) for _ in range(2 * GG)],
            pltpu.VMEM_SHARED((NP, d), jnp.float32),
            pltpu.SemaphoreType.DMA,
        ],
    )
    def edge_kernel(hs_hbm, src_hbm, dst_hbm, zeros_hbm, out_hbm,
                    src_v, dst_v, rows, acc, sem):
        c = lax.axis_index("c")
        s = lax.axis_index("s")
        tid = c * NS + s
        r0 = s * RPT
        @pl.when(c == 0)
        def _():
            pltpu.sync_copy(hs_hbm.at[pl.ds(r0, RPT), :], acc.at[pl.ds(r0, RPT), :])

        @pl.when(c != 0)
        def _():
            pltpu.sync_copy(zeros_hbm.at[pl.ds(r0, RPT), :], acc.at[pl.ds(r0, RPT), :])

        pltpu.sync_copy(src_hbm.at[tid], src_v)
        pltpu.sync_copy(dst_hbm.at[tid], dst_v)
        plsc.subcore_barrier()

        for b in range(GG):
            pltpu.async_copy(hs_hbm.at[src_v.at[b]], rows[b], sem)

        def body(gp, carry):
            for h in (0, 1):
                g = 2 * gp + h
                j0 = g * GG
                bs = [h * GG + b for b in range(GG)]
                for b in bs:
                    pltpu.make_async_copy(hs_hbm.at[src_v.at[0]], rows[b], sem).wait()
                @pl.when(g + 1 < G)
                def _(g=g, h=h):
                    for b in range(GG):
                        pltpu.async_copy(
                            hs_hbm.at[src_v.at[(g + 1) * GG + b]],
                            rows[(1 - h) * GG + b], sem)
                for i, b in enumerate(bs):
                    pltpu.sync_copy(rows[b], acc.at[dst_v.at[j0 + i]], add=True)
            return carry

        lax.fori_loop(0, G // 2, body, 0)
        plsc.subcore_barrier()
        pltpu.sync_copy(acc.at[pl.ds(r0, RPT), :], out_hbm.at[c, pl.ds(r0, RPT), :])

    return edge_kernel


_deg_kernel = _make_deg_kernel()
_edge_kernel16 = _make_edge_kernel(D_HID)


def _tc_stage1(feats_ref, w1_ref, degp_ref, hs1_ref, dinv_ref):
    deg = degp_ref[0] + degp_ref[1] + 1.0
    dinv = lax.rsqrt(deg)
    dinv_ref[...] = dinv
    h = jnp.dot(feats_ref[...], w1_ref[...], preferred_element_type=jnp.float32)
    hs1_ref[...] = h * dinv[:, 0:1]


def _tc_stage2(p_ref, dinv_ref, b1_ref, h1s_ref):
    dinv1 = dinv_ref[:, 0:1]
    h1 = jnp.maximum((p_ref[0] + p_ref[1]) * dinv1 + b1_ref[...], 0.0)
    h1s_ref[...] = h1 * dinv1


def _tc_stage3(p_ref, dinv_ref, w2_ref, b2_ref, out_ref):
    agg = (p_ref[0] + p_ref[1]) * dinv_ref[:, 0:1]
    out_ref[...] = (
        jnp.dot(agg, w2_ref[...], preferred_element_type=jnp.float32) + b2_ref[...]
    )


def kernel(feats, edge_index, W1, b1, W2, b2):
    f32 = jnp.float32
    feats_p = jnp.pad(feats, ((0, NP - N), (0, 0)))
    src = jnp.pad(edge_index[0], (0, E_PAD - E))
    dst = jnp.pad(edge_index[1], (0, E_PAD - E), constant_values=N)
    src_t = src.reshape(NW, K, EC)
    dst_t = dst.reshape(NW, K, EC)
    ones16 = jnp.ones((EC, 16), f32)
    zeros16 = jnp.zeros((NP, 16), f32)
    w2p = jnp.pad(W2, ((0, 0), (0, D2 - N_CLS)))
    b1r = b1.reshape(1, D_HID)
    b2r = jnp.pad(b2, (0, D2 - N_CLS)).reshape(1, D2)

    degp = _deg_kernel(dst_t, ones16, zeros16)

    hs1, dinv = pl.pallas_call(
        _tc_stage1,
        out_shape=[
            jax.ShapeDtypeStruct((NP, D_HID), f32),
            jax.ShapeDtypeStruct((NP, 16), f32),
        ],
    )(feats_p, W1, degp)

    p1 = _edge_kernel16(hs1, src_t, dst_t, zeros16)

    h1s = pl.pallas_call(
        _tc_stage2,
        out_shape=jax.ShapeDtypeStruct((NP, D_HID), f32),
    )(p1, dinv, b1r)

    p2 = _edge_kernel16(h1s, src_t, dst_t, zeros16)

    out = pl.pallas_call(
        _tc_stage3,
        out_shape=jax.ShapeDtypeStruct((NP, D2), f32),
    )(p2, dinv, w2p, b2r)

    return out[:N, :N_CLS]
